# baseline (speedup 1.0000x reference)
import functools

import jax
import jax.numpy as jnp
from jax import lax
from jax.experimental import pallas as pl
from jax.experimental.pallas import tpu as pltpu
from jax.experimental.pallas import tpu_sc as plsc

_K = 64
_S = 4


def _thresh_body(x_ref, thr_ref, cut_ref, *, k):
    x = x_ref[...]
    r_rows, n = x.shape
    b = lax.bitcast_convert_type(x, jnp.int32)
    key = jnp.where(b >= 0, b, b ^ jnp.int32(0x7FFFFFFF))

    i32 = jnp.int32
    kc = key.reshape(r_rows, k, n // k)
    cmax = jnp.max(kc, axis=2)
    lb = jnp.min(cmax, axis=1, keepdims=True)
    ub = jnp.max(cmax, axis=1, keepdims=True)

    def vcond(carry):
        lo, hi = carry
        return jnp.any(lo < hi)

    def vstep(carry):
        lo, hi = carry
        mid = (lo >> 1) + (hi >> 1) + (lo & hi & 1)
        cnt = jnp.sum((key > mid).astype(i32), axis=1, keepdims=True)
        ge = cnt >= k
        return jnp.where(ge, mid + 1, lo), jnp.where(ge, hi, mid)

    lo, _ = lax.while_loop(vcond, vstep, (lb, ub))
    v = lo
    gt = key > v
    eq = key == v
    cgt = jnp.sum(gt.astype(i32), axis=1, keepdims=True)
    ceq = jnp.sum(eq.astype(i32), axis=1, keepdims=True)
    need = k - cgt

    idx = lax.broadcasted_iota(i32, (r_rows, n), 1)
    tie = ceq > need
    lo2 = jnp.where(tie, 0, n - 1)
    hi2 = jnp.full((r_rows, 1), n - 1, i32)

    def icond(carry):
        lo2, hi2 = carry
        return jnp.any(lo2 < hi2)

    def istep(carry):
        lo2, hi2 = carry
        mid = (lo2 + hi2) >> 1
        cnt = jnp.sum((eq & (idx <= mid)).astype(i32), axis=1, keepdims=True)
        ge = cnt >= need
        return jnp.where(ge, lo2, mid + 1), jnp.where(ge, mid, hi2)

    lo2, _ = lax.while_loop(icond, istep, (lo2, hi2))
    thr_ref[...] = jnp.broadcast_to(v, (r_rows, 128))
    cut_ref[...] = jnp.broadcast_to(lo2, (r_rows, 128))


def _thresholds(x, k, rows_per_block):
    bsz, n = x.shape
    grid = bsz // rows_per_block
    body = functools.partial(_thresh_body, k=k)
    return pl.pallas_call(
        body,
        grid=(grid,),
        in_specs=[pl.BlockSpec((rows_per_block, n), lambda i: (i, 0))],
        out_specs=[
            pl.BlockSpec((rows_per_block, 128), lambda i: (i, 0)),
            pl.BlockSpec((rows_per_block, 128), lambda i: (i, 0)),
        ],
        out_shape=[
            jax.ShapeDtypeStruct((bsz, 128), jnp.int32),
            jax.ShapeDtypeStruct((bsz, 128), jnp.int32),
        ],
    )(x)


def _sc_expand(x, thr, cut, s):
    bsz, n = x.shape
    info = plsc.get_sparse_core_info()
    nw = info.num_cores * info.num_subcores
    rows_w = bsz // nw
    nvec = n // 16
    unroll = 4
    mesh = plsc.VectorSubcoreMesh(core_axis_name="c", subcore_axis_name="s")

    @functools.partial(
        pl.kernel,
        mesh=mesh,
        out_type=jax.ShapeDtypeStruct((s, bsz, n), jnp.float32),
        scratch_types=[
            pltpu.VMEM((n,), jnp.float32),
            pltpu.VMEM((n,), jnp.float32),
            pltpu.VMEM((128,), jnp.int32),
            pltpu.VMEM((128,), jnp.int32),
        ],
    )
    def k(x_hbm, thr_hbm, cut_hbm, out_hbm, xrow, mrow, thrv, cutv):
        wid = lax.axis_index("s") * info.num_cores + lax.axis_index("c")
        lane = lax.iota(jnp.int32, 16)
        for r in range(rows_w):
            row = wid * rows_w + r
            pltpu.sync_copy(x_hbm.at[row], xrow)
            pltpu.sync_copy(thr_hbm.at[row], thrv)
            pltpu.sync_copy(cut_hbm.at[row], cutv)
            tv = thrv[pl.ds(0, 16)]
            cv = cutv[pl.ds(0, 16)]

            def body(j, carry):
                for u in range(unroll):
                    base = (j * unroll + u) * 16
                    xv = xrow[pl.ds(base, 16)]
                    bv = lax.bitcast_convert_type(xv, jnp.int32)
                    kv = jnp.where(bv >= 0, bv, bv ^ jnp.int32(0x7FFFFFFF))
                    iv = base + lane
                    m = (kv > tv) | ((kv == tv) & (iv <= cv))
                    mrow[pl.ds(base, 16)] = jnp.where(
                        m, jnp.float32(1.0), jnp.float32(0.0)
                    )
                return carry

            lax.fori_loop(0, nvec // unroll, body, jnp.int32(0))
            for si in range(s):
                pltpu.sync_copy(mrow, out_hbm.at[si, row])

    return k(x, thr, cut)


def kernel(logits):
    bsz, n, _ = logits.shape
    x = jnp.squeeze(logits, axis=-1)
    rows_per_block = 32 if bsz % 32 == 0 else bsz
    if bsz % 64 == 0:
        h = bsz // 2
        xa, xb = x[:h], x[h:]
        thra, cuta = _thresholds(xa, _K, rows_per_block)
        thrb, cutb = _thresholds(xb, _K, rows_per_block)
        outa = _sc_expand(xa, thra, cuta, _S)
        outb = _sc_expand(xb, thrb, cutb, _S)
        out = jnp.concatenate([outa, outb], axis=1)
    else:
        thr, cut = _thresholds(x, _K, rows_per_block)
        out = _sc_expand(x, thr, cut, _S)
    return out.reshape(_S, bsz, n, 1)


# speedup vs baseline: 1.0349x; 1.0349x over previous
import functools

import jax
import jax.numpy as jnp
from jax import lax
from jax.experimental import pallas as pl
from jax.experimental.pallas import tpu as pltpu
from jax.experimental.pallas import tpu_sc as plsc

_K = 64
_S = 4


def _thresh_body(x_ref, thr_ref, cut_ref, *, k):
    x = x_ref[...]
    r_rows, n = x.shape
    b = lax.bitcast_convert_type(x, jnp.int32)
    key = jnp.where(b >= 0, b, b ^ jnp.int32(0x7FFFFFFF))

    i32 = jnp.int32
    kc = key.reshape(r_rows, k, n // k)
    cmax = jnp.max(kc, axis=2)
    lb = jnp.min(cmax, axis=1, keepdims=True)
    ub = jnp.max(cmax, axis=1, keepdims=True)

    def vcond(carry):
        lo, hi = carry
        return jnp.any(lo < hi)

    def vstep(carry):
        lo, hi = carry
        mid = (lo >> 1) + (hi >> 1) + (lo & hi & 1)
        cnt = jnp.sum((key > mid).astype(i32), axis=1, keepdims=True)
        ge = cnt >= k
        return jnp.where(ge, mid + 1, lo), jnp.where(ge, hi, mid)

    lo, _ = lax.while_loop(vcond, vstep, (lb, ub))
    v = lo
    gt = key > v
    eq = key == v
    cgt = jnp.sum(gt.astype(i32), axis=1, keepdims=True)
    ceq = jnp.sum(eq.astype(i32), axis=1, keepdims=True)
    need = k - cgt

    idx = lax.broadcasted_iota(i32, (r_rows, n), 1)
    tie = ceq > need
    lo2 = jnp.where(tie, 0, n - 1)
    hi2 = jnp.full((r_rows, 1), n - 1, i32)

    def icond(carry):
        lo2, hi2 = carry
        return jnp.any(lo2 < hi2)

    def istep(carry):
        lo2, hi2 = carry
        mid = (lo2 + hi2) >> 1
        cnt = jnp.sum((eq & (idx <= mid)).astype(i32), axis=1, keepdims=True)
        ge = cnt >= need
        return jnp.where(ge, lo2, mid + 1), jnp.where(ge, mid, hi2)

    lo2, _ = lax.while_loop(icond, istep, (lo2, hi2))
    thr_ref[...] = jnp.broadcast_to(v, (r_rows, 128))
    cut_ref[...] = jnp.broadcast_to(lo2, (r_rows, 128))


def _thresholds(x, k, rows_per_block):
    bsz, n = x.shape
    grid = bsz // rows_per_block
    body = functools.partial(_thresh_body, k=k)
    return pl.pallas_call(
        body,
        grid=(grid,),
        in_specs=[pl.BlockSpec((rows_per_block, n), lambda i: (i, 0))],
        out_specs=[
            pl.BlockSpec((rows_per_block, 128), lambda i: (i, 0)),
            pl.BlockSpec((rows_per_block, 128), lambda i: (i, 0)),
        ],
        out_shape=[
            jax.ShapeDtypeStruct((bsz, 128), jnp.int32),
            jax.ShapeDtypeStruct((bsz, 128), jnp.int32),
        ],
    )(x)


def _sc_expand(x, thr, cut, s):
    bsz, n = x.shape
    info = plsc.get_sparse_core_info()
    nw = info.num_cores * info.num_subcores
    rows_w = bsz // nw
    nvec = n // 16
    unroll = 4
    mesh = plsc.VectorSubcoreMesh(core_axis_name="c", subcore_axis_name="s")

    @functools.partial(
        pl.kernel,
        mesh=mesh,
        out_type=jax.ShapeDtypeStruct((s, bsz, n), jnp.float32),
        scratch_types=[
            pltpu.VMEM((n,), jnp.float32),
            pltpu.VMEM((2, n), jnp.float32),
            pltpu.VMEM((rows_w, 128), jnp.int32),
            pltpu.VMEM((rows_w, 128), jnp.int32),
            pltpu.SemaphoreType.DMA,
        ],
    )
    def k(x_hbm, thr_hbm, cut_hbm, out_hbm, xrow, mrow, thrv, cutv, osem):
        wid = lax.axis_index("s") * info.num_cores + lax.axis_index("c")
        base_row = wid * rows_w
        lane = lax.iota(jnp.int32, 16)
        pltpu.sync_copy(thr_hbm.at[pl.ds(base_row, rows_w)], thrv)
        pltpu.sync_copy(cut_hbm.at[pl.ds(base_row, rows_w)], cutv)
        out_dma = [None] * rows_w
        for r in range(rows_w):
            row = base_row + r
            pltpu.sync_copy(x_hbm.at[row], xrow)
            tv = thrv[r, pl.ds(0, 16)]
            cv = cutv[r, pl.ds(0, 16)]
            mbuf = mrow.at[r % 2]
            if r >= 2:
                for h in out_dma[r - 2]:
                    h.wait()

            def body(j, carry):
                for u in range(unroll):
                    base = (j * unroll + u) * 16
                    xv = xrow[pl.ds(base, 16)]
                    bv = lax.bitcast_convert_type(xv, jnp.int32)
                    kv = jnp.where(bv >= 0, bv, bv ^ jnp.int32(0x7FFFFFFF))
                    iv = base + lane
                    m = (kv > tv) | ((kv == tv) & (iv <= cv))
                    mbuf[pl.ds(base, 16)] = jnp.where(
                        m, jnp.float32(1.0), jnp.float32(0.0)
                    )
                return carry

            lax.fori_loop(0, nvec // unroll, body, jnp.int32(0))
            out_dma[r] = [
                pltpu.async_copy(mbuf, out_hbm.at[si, row], osem)
                for si in range(s)
            ]
        for r in range(max(0, rows_w - 2), rows_w):
            for h in out_dma[r]:
                h.wait()

    return k(x, thr, cut)


def kernel(logits):
    bsz, n, _ = logits.shape
    x = jnp.squeeze(logits, axis=-1)
    rows_per_block = 32 if bsz % 32 == 0 else bsz
    if False:
        h = bsz // 2
        xa, xb = x[:h], x[h:]
        thra, cuta = _thresholds(xa, _K, rows_per_block)
        thrb, cutb = _thresholds(xb, _K, rows_per_block)
        outa = _sc_expand(xa, thra, cuta, _S)
        outb = _sc_expand(xb, thrb, cutb, _S)
        out = jnp.concatenate([outa, outb], axis=1)
    else:
        thr, cut = _thresholds(x, _K, rows_per_block)
        out = _sc_expand(x, thr, cut, _S)
    return out.reshape(_S, bsz, n, 1)
